# Initial kernel scaffold; baseline (speedup 1.0000x reference)
import jax
import jax.numpy as jnp
from jax.experimental import pallas as pl


def kernel(x, gate_w1, gate_b1, gate_w2, gate_b2, expert_w1, expert_b1, expert_w2, expert_b2, proj_w, blend):
    raise NotImplementedError("write your pallas kernel here")


# speedup vs baseline: 1.2203x; 1.2203x over previous
import jax
import jax.numpy as jnp
from jax.experimental import pallas as pl
from jax.experimental.pallas import tpu as pltpu

THRESH = 0.1


def _fused_body(x_ref, gw1_ref, gb1_ref, gw2_ref, gb2_ref,
                ew1_ref, eb1_ref, ew2_ref, eb2_ref, pw_ref, alpha_ref,
                out_ref, gwout_ref, sc_s):
    E = pl.num_programs(0)
    e = pl.program_id(0)
    mb = pl.program_id(1)
    BM = x_ref.shape[0]
    rows = pl.ds(mb * BM, BM)
    xb = x_ref[...]

    @pl.when(e == 0)
    def _gate():
        h = jnp.dot(xb, gw1_ref[...], preferred_element_type=jnp.float32)
        h = h + gb1_ref[...]
        h = h * jax.nn.sigmoid(h)
        logits = jnp.dot(h, gw2_ref[...], preferred_element_type=jnp.float32)
        logits = logits + gb2_ref[...]
        m = jnp.max(logits, axis=1, keepdims=True)
        p = jnp.exp(logits - m)
        gw = p / jnp.sum(p, axis=1, keepdims=True)
        gwout_ref[rows, :] = gw
        lane = jax.lax.broadcasted_iota(jnp.int32, (BM, E), 1)
        cols = []
        for ee in range(E):
            ge = gw[:, ee:ee + 1]
            gt = jnp.sum((gw > ge).astype(jnp.int32), axis=1, keepdims=True)
            eqb = jnp.sum(((gw == ge) & (lane < ee)).astype(jnp.int32),
                          axis=1, keepdims=True)
            in_top2 = (gt + eqb) < 2
            fire = in_top2 & (ge > THRESH)
            cols.append(jnp.where(fire, ge, 0.0))
        w = jnp.concatenate(cols, axis=1)
        sc_s[rows, 0:E] = w
        tw = jnp.sum(w, axis=1, keepdims=True)
        fired = tw > 0.0
        stw = jnp.where(fired, tw, 1.0)
        a = alpha_ref[0, 0]
        sc_s[rows, E:E + 1] = a / stw
        sc_s[rows, E + 1:E + 2] = jnp.where(fired, 1.0 - a, 1.0)

    h1 = jnp.dot(xb, ew1_ref[0], preferred_element_type=jnp.float32)
    h1 = h1 + eb1_ref[0]
    h1 = h1 * jax.nn.sigmoid(h1)
    eo = jnp.dot(h1, ew2_ref[0], preferred_element_type=jnp.float32)
    eo = eo + eb2_ref[0]
    po = jnp.dot(eo, pw_ref[0], preferred_element_type=jnp.float32)
    lane = jax.lax.broadcasted_iota(jnp.int32, (BM, E), 1)
    we = jnp.sum(jnp.where(lane == e, sc_s[rows, 0:E], 0.0), axis=1,
                 keepdims=True)
    contrib = we * po

    @pl.when(e == 0)
    def _init():
        out_ref[rows, :] = contrib

    @pl.when(e != 0)
    def _acc():
        out_ref[rows, :] = out_ref[rows, :] + contrib

    @pl.when(e == E - 1)
    def _final():
        out_ref[rows, :] = (sc_s[rows, E:E + 1] * out_ref[rows, :]
                            + sc_s[rows, E + 1:E + 2] * xb)


def kernel(x, gate_w1, gate_b1, gate_w2, gate_b2,
           expert_w1, expert_b1, expert_w2, expert_b2, proj_w, blend):
    B, H = x.shape
    H2 = gate_w1.shape[1]
    E, _, F = expert_w1.shape
    BM = 256
    MB = B // BM
    alpha = jax.nn.sigmoid(blend).reshape(1, 1).astype(jnp.float32)

    out, gate_weights = pl.pallas_call(
        _fused_body,
        grid=(E, MB),
        in_specs=[
            pl.BlockSpec((BM, H), lambda e, mb: (mb, 0)),
            pl.BlockSpec((H, H2), lambda e, mb: (0, 0)),
            pl.BlockSpec((1, H2), lambda e, mb: (0, 0)),
            pl.BlockSpec((H2, E), lambda e, mb: (0, 0)),
            pl.BlockSpec((1, E), lambda e, mb: (0, 0)),
            pl.BlockSpec((1, H, F), lambda e, mb: (e, 0, 0)),
            pl.BlockSpec((1, 1, F), lambda e, mb: (e, 0, 0)),
            pl.BlockSpec((1, F, H), lambda e, mb: (e, 0, 0)),
            pl.BlockSpec((1, 1, H), lambda e, mb: (e, 0, 0)),
            pl.BlockSpec((1, H, H), lambda e, mb: (e, 0, 0)),
            pl.BlockSpec(memory_space=pltpu.SMEM),
        ],
        out_specs=[
            pl.BlockSpec((B, H), lambda e, mb: (0, 0)),
            pl.BlockSpec((B, E), lambda e, mb: (0, 0)),
        ],
        out_shape=[
            jax.ShapeDtypeStruct((B, H), jnp.float32),
            jax.ShapeDtypeStruct((B, E), jnp.float32),
        ],
        scratch_shapes=[
            pltpu.VMEM((B, 16), jnp.float32),
        ],
        compiler_params=pltpu.CompilerParams(
            vmem_limit_bytes=100 * 1024 * 1024,
        ),
    )(x, gate_w1, gate_b1.reshape(1, H2), gate_w2, gate_b2.reshape(1, E),
      expert_w1, expert_b1.reshape(E, 1, F), expert_w2,
      expert_b2.reshape(E, 1, H), proj_w, alpha)
    return out, gate_weights
